# baseline (speedup 1.0000x reference)
import functools

import numpy as np
import jax
import jax.numpy as jnp
from jax import lax
from jax.experimental import pallas as pl
from jax.experimental.pallas import tpu as pltpu
from jax.experimental.pallas import tpu_sc as plsc

DIM = 64
BATCH = 16384

NUM_CORES = 2
NUM_SUBCORES = 16
NUM_WORKERS = NUM_CORES * NUM_SUBCORES
B_PER_W = BATCH // NUM_WORKERS
CHUNK = 128
N_CHUNKS = B_PER_W // CHUNK
GROUPS = B_PER_W // 16

_TAKE_DNUMS = lax.GatherDimensionNumbers(
    offset_dims=(), collapsed_slice_dims=(0,), start_index_map=(0,))


def _lane_permute(v, perm):
    return lax.gather(v, perm[:, None], dimension_numbers=_TAKE_DNUMS,
                      slice_sizes=(1,),
                      mode=lax.GatherScatterMode.PROMISE_IN_BOUNDS)


def _lane_sum(v, perms):
    for perm in perms:
        v = v + _lane_permute(v, perm)
    return v


def _rsqrt_nr(x):
    xi = lax.bitcast_convert_type(x, jnp.int32)
    yi = 0x5F3759DF - lax.shift_right_arithmetic(xi, 1)
    y = lax.bitcast_convert_type(yi, jnp.float32)
    for _ in range(2):
        y = y * (1.5 - 0.5 * x * y * y)
    return y


def _transe_sc(batch_h, batch_t, batch_r, ent_emb, rel_emb):
    mesh = plsc.VectorSubcoreMesh(core_axis_name="c", subcore_axis_name="s")

    @functools.partial(
        pl.kernel,
        mesh=mesh,
        out_type=jax.ShapeDtypeStruct((BATCH,), jnp.float32),
        compiler_params=pltpu.CompilerParams(use_tc_tiling_on_sc=False),
        scratch_types=[
            pltpu.VMEM((N_CHUNKS, CHUNK), jnp.int32),
            pltpu.VMEM((N_CHUNKS, CHUNK), jnp.int32),
            pltpu.VMEM((N_CHUNKS, CHUNK), jnp.int32),
            pltpu.VMEM((B_PER_W, DIM), jnp.float32),
            pltpu.VMEM((B_PER_W, DIM), jnp.float32),
            pltpu.VMEM((B_PER_W, DIM), jnp.float32),
            pltpu.VMEM((B_PER_W,), jnp.float32),
            pltpu.SemaphoreType.DMA,
        ],
    )
    def k(bh_hbm, bt_hbm, br_hbm, ent_hbm, rel_hbm, out_hbm,
          idx_h, idx_t, idx_r, h_rows, t_rows, r_rows, out_v, sem):
        wid = lax.axis_index("s") * NUM_CORES + lax.axis_index("c")
        base = wid * B_PER_W

        for c in range(N_CHUNKS):
            off = base + c * CHUNK
            pltpu.sync_copy(bh_hbm.at[pl.ds(off, CHUNK)], idx_h.at[c])
            pltpu.sync_copy(bt_hbm.at[pl.ds(off, CHUNK)], idx_t.at[c])
            pltpu.sync_copy(br_hbm.at[pl.ds(off, CHUNK)], idx_r.at[c])

        copies = []
        for c in range(N_CHUNKS):
            dst = pl.ds(c * CHUNK, CHUNK)
            copies.append(pltpu.async_copy(ent_hbm.at[idx_h.at[c]], h_rows.at[dst], sem))
            copies.append(pltpu.async_copy(ent_hbm.at[idx_t.at[c]], t_rows.at[dst], sem))
            copies.append(pltpu.async_copy(rel_hbm.at[idx_r.at[c]], r_rows.at[dst], sem))
        for cp in copies:
            cp.wait()

        iota16 = lax.iota(jnp.int32, 16)
        perms = [lax.bitwise_xor(iota16, jnp.int32(k)) for k in (1, 2, 4, 8)]

        def group_body(g, _):
            acc = jnp.zeros((16,), jnp.float32)
            for j in range(16):
                i = g * 16 + j
                hv = [h_rows[i, pl.ds(16 * q, 16)] for q in range(4)]
                tv = [t_rows[i, pl.ds(16 * q, 16)] for q in range(4)]
                rv = [r_rows[i, pl.ds(16 * q, 16)] for q in range(4)]

                def inv_norm(vs):
                    ssq = ((vs[0] * vs[0] + vs[1] * vs[1])
                           + (vs[2] * vs[2] + vs[3] * vs[3]))
                    return _rsqrt_nr(jnp.maximum(_lane_sum(ssq, perms), 1e-24))

                ih = inv_norm(hv)
                it = inv_norm(tv)
                ir = inv_norm(rv)

                s = jnp.zeros((16,), jnp.float32)
                for q in range(4):
                    s = s + jnp.abs(hv[q] * ih + rv[q] * ir - tv[q] * it)
                score = _lane_sum(s, perms)
                acc = jnp.where(iota16 == j, score, acc)
            out_v[pl.ds(g * 16, 16)] = acc
            return 0

        lax.fori_loop(0, GROUPS, group_body, 0)

        pltpu.sync_copy(out_v, out_hbm.at[pl.ds(base, B_PER_W)])

    return k(batch_h, batch_t, batch_r, ent_emb, rel_emb)


def kernel(batch_h, batch_t, batch_r, ent_emb, rel_emb):
    return _transe_sc(batch_h, batch_t, batch_r, ent_emb, rel_emb)


# speedup vs baseline: 1.4703x; 1.4703x over previous
import functools

import numpy as np
import jax
import jax.numpy as jnp
from jax import lax
from jax.experimental import pallas as pl
from jax.experimental.pallas import tpu as pltpu
from jax.experimental.pallas import tpu_sc as plsc

ENT_TOT = 1000000
REL_TOT = 1000
DIM = 64
SUB = 8
BATCH = 16384

NUM_CORES = 2
NUM_SUBCORES = 16
NUM_WORKERS = NUM_CORES * NUM_SUBCORES
B_PER_W = BATCH // NUM_WORKERS
CHUNK = 16
N_CHUNKS = B_PER_W // CHUNK
PAIRS = N_CHUNKS // 2

_TAKE_DNUMS = lax.GatherDimensionNumbers(
    offset_dims=(), collapsed_slice_dims=(0,), start_index_map=(0,))


def _lane_permute(v, perm):
    return lax.gather(v, perm[:, None], dimension_numbers=_TAKE_DNUMS,
                      slice_sizes=(1,),
                      mode=lax.GatherScatterMode.PROMISE_IN_BOUNDS)


def _lane_sum(v, perms):
    for perm in perms:
        v = v + _lane_permute(v, perm)
    return v


def _rsqrt_nr(x):
    xi = lax.bitcast_convert_type(x, jnp.int32)
    yi = 0x5F3759DF - lax.shift_right_arithmetic(xi, 1)
    y = lax.bitcast_convert_type(yi, jnp.float32)
    for _ in range(2):
        y = y * (1.5 - 0.5 * x * y * y)
    return y


def _transe_sc(batch_h, batch_t, batch_r, ent_emb, rel_emb):
    mesh = plsc.VectorSubcoreMesh(core_axis_name="c", subcore_axis_name="s")

    @functools.partial(
        pl.kernel,
        mesh=mesh,
        out_type=jax.ShapeDtypeStruct((BATCH,), jnp.float32),
        scratch_types=[
            pltpu.VMEM((B_PER_W,), jnp.int32),
            pltpu.VMEM((B_PER_W,), jnp.int32),
            pltpu.VMEM((B_PER_W,), jnp.int32),
            pltpu.VMEM((2, CHUNK, SUB, DIM), jnp.float32),
            pltpu.VMEM((2, CHUNK, SUB, DIM), jnp.float32),
            pltpu.VMEM((2, CHUNK, SUB, DIM), jnp.float32),
            pltpu.VMEM((B_PER_W,), jnp.float32),
            pltpu.SemaphoreType.DMA,
            pltpu.SemaphoreType.DMA,
        ],
    )
    def k(bh_hbm, bt_hbm, br_hbm, ent_hbm, rel_hbm, out_hbm,
          idx_h, idx_t, idx_r, h_buf, t_buf, r_buf, out_v, sem0, sem1):
        wid = lax.axis_index("s") * NUM_CORES + lax.axis_index("c")
        base = wid * B_PER_W

        ent_t = ent_hbm.reshape(ENT_TOT // SUB, SUB, DIM)
        rel_t = rel_hbm.reshape(REL_TOT // SUB, SUB, DIM)

        pltpu.sync_copy(bh_hbm.at[pl.ds(base, B_PER_W)], idx_h)
        pltpu.sync_copy(bt_hbm.at[pl.ds(base, B_PER_W)], idx_t)
        pltpu.sync_copy(br_hbm.at[pl.ds(base, B_PER_W)], idx_r)

        sems = (sem0, sem1)
        iota16 = lax.iota(jnp.int32, 16)
        perms = [lax.bitwise_xor(iota16, jnp.int32(kk)) for kk in (1, 2, 4, 8)]

        def fire(c, b):
            hvec = idx_h[pl.ds(c * CHUNK, CHUNK)]
            tvec = idx_t[pl.ds(c * CHUNK, CHUNK)]
            rvec = idx_r[pl.ds(c * CHUNK, CHUNK)]
            for vec, tab, buf in ((hvec, ent_t, h_buf),
                                  (tvec, ent_t, t_buf),
                                  (rvec, rel_t, r_buf)):
                cps = []
                for j in range(CHUNK):
                    i = vec[j]
                    ts = lax.shift_right_logical(i, 3)
                    ss = lax.bitwise_and(i, 7)
                    cps.append(pltpu.async_copy(
                        tab.at[ts, ss], buf.at[b, j, ss], sems[b]))
                for cp in cps:
                    cp.wait()

        def wait(b):
            del b

        def compute(c, b):
            hvec = idx_h[pl.ds(c * CHUNK, CHUNK)]
            tvec = idx_t[pl.ds(c * CHUNK, CHUNK)]
            rvec = idx_r[pl.ds(c * CHUNK, CHUNK)]
            acc = jnp.zeros((16,), jnp.float32)
            for j in range(CHUNK):
                sh = lax.bitwise_and(hvec[j], 7)
                st = lax.bitwise_and(tvec[j], 7)
                sr = lax.bitwise_and(rvec[j], 7)
                hv = [h_buf[b, j, sh, pl.ds(16 * q, 16)] for q in range(4)]
                tv = [t_buf[b, j, st, pl.ds(16 * q, 16)] for q in range(4)]
                rv = [r_buf[b, j, sr, pl.ds(16 * q, 16)] for q in range(4)]

                def inv_norm(vs):
                    ssq = ((vs[0] * vs[0] + vs[1] * vs[1])
                           + (vs[2] * vs[2] + vs[3] * vs[3]))
                    return _rsqrt_nr(jnp.maximum(_lane_sum(ssq, perms), 1e-24))

                ih = inv_norm(hv)
                it = inv_norm(tv)
                ir = inv_norm(rv)

                s = jnp.zeros((16,), jnp.float32)
                for q in range(4):
                    s = s + jnp.abs(hv[q] * ih + rv[q] * ir - tv[q] * it)
                score = _lane_sum(s, perms)
                acc = jnp.where(iota16 == j, score, acc)
            out_v[pl.ds(c * CHUNK, 16)] = acc

        def chunk_body(c, _):
            fire(c, 0)
            wait(0)
            compute(c, 0)
            return 0

        lax.fori_loop(0, N_CHUNKS, chunk_body, 0)

        pltpu.sync_copy(out_v, out_hbm.at[pl.ds(base, B_PER_W)])

    return k(batch_h, batch_t, batch_r, ent_emb, rel_emb)


def kernel(batch_h, batch_t, batch_r, ent_emb, rel_emb):
    return _transe_sc(batch_h, batch_t, batch_r, ent_emb, rel_emb)


# speedup vs baseline: 1.5465x; 1.0518x over previous
import functools

import numpy as np
import jax
import jax.numpy as jnp
from jax import lax
from jax.experimental import pallas as pl
from jax.experimental.pallas import tpu as pltpu
from jax.experimental.pallas import tpu_sc as plsc

ENT_TOT = 1000000
REL_TOT = 1000
DIM = 64
SUB = 8
BATCH = 16384

NUM_CORES = 2
NUM_SUBCORES = 16
NUM_WORKERS = NUM_CORES * NUM_SUBCORES
B_PER_W = BATCH // NUM_WORKERS
CHUNK = 16
N_CHUNKS = B_PER_W // CHUNK
PAIRS = N_CHUNKS // 2

_TAKE_DNUMS = lax.GatherDimensionNumbers(
    offset_dims=(), collapsed_slice_dims=(0,), start_index_map=(0,))


def _lane_permute(v, perm):
    return lax.gather(v, perm[:, None], dimension_numbers=_TAKE_DNUMS,
                      slice_sizes=(1,),
                      mode=lax.GatherScatterMode.PROMISE_IN_BOUNDS)


def _lane_sum(v, perms):
    for perm in perms:
        v = v + _lane_permute(v, perm)
    return v


def _rsqrt_nr(x):
    xi = lax.bitcast_convert_type(x, jnp.int32)
    yi = 0x5F3759DF - lax.shift_right_arithmetic(xi, 1)
    y = lax.bitcast_convert_type(yi, jnp.float32)
    for _ in range(2):
        y = y * (1.5 - 0.5 * x * y * y)
    return y


def _transe_sc(batch_h, batch_t, batch_r, ent_emb, rel_emb):
    mesh = plsc.VectorSubcoreMesh(core_axis_name="c", subcore_axis_name="s")

    @functools.partial(
        pl.kernel,
        mesh=mesh,
        out_type=jax.ShapeDtypeStruct((BATCH,), jnp.float32),
        scratch_types=[
            pltpu.VMEM((B_PER_W,), jnp.int32),
            pltpu.VMEM((B_PER_W,), jnp.int32),
            pltpu.VMEM((B_PER_W,), jnp.int32),
            pltpu.VMEM((2, CHUNK, SUB, DIM), jnp.float32),
            pltpu.VMEM((2, CHUNK, SUB, DIM), jnp.float32),
            pltpu.VMEM((2, CHUNK, SUB, DIM), jnp.float32),
            pltpu.VMEM((B_PER_W,), jnp.float32),
            pltpu.SemaphoreType.DMA,
            pltpu.SemaphoreType.DMA,
        ],
    )
    def k(bh_hbm, bt_hbm, br_hbm, ent_hbm, rel_hbm, out_hbm,
          idx_h, idx_t, idx_r, h_buf, t_buf, r_buf, out_v, sem0, sem1):
        wid = lax.axis_index("s") * NUM_CORES + lax.axis_index("c")
        base = wid * B_PER_W

        ent_t = ent_hbm.reshape(ENT_TOT // SUB, SUB, DIM)
        rel_t = rel_hbm.reshape(REL_TOT // SUB, SUB, DIM)

        pltpu.sync_copy(bh_hbm.at[pl.ds(base, B_PER_W)], idx_h)
        pltpu.sync_copy(bt_hbm.at[pl.ds(base, B_PER_W)], idx_t)
        pltpu.sync_copy(br_hbm.at[pl.ds(base, B_PER_W)], idx_r)

        sems = (sem0, sem1)
        iota16 = lax.iota(jnp.int32, 16)
        perms = [lax.bitwise_xor(iota16, jnp.int32(kk)) for kk in (1, 2, 4, 8)]

        def fire(c, b):
            hvec = idx_h[pl.ds(c * CHUNK, CHUNK)]
            tvec = idx_t[pl.ds(c * CHUNK, CHUNK)]
            rvec = idx_r[pl.ds(c * CHUNK, CHUNK)]
            cps = []
            for vec, tab, buf in ((hvec, ent_t, h_buf),
                                  (tvec, ent_t, t_buf),
                                  (rvec, rel_t, r_buf)):
                for j in range(CHUNK):
                    i = vec[j]
                    ts = lax.shift_right_logical(i, 3)
                    ss = lax.bitwise_and(i, 7)
                    cps.append(pltpu.async_copy(
                        tab.at[ts, ss], buf.at[b, j, ss], sems[b]))
            return cps

        def compute(c, b):
            hvec = idx_h[pl.ds(c * CHUNK, CHUNK)]
            tvec = idx_t[pl.ds(c * CHUNK, CHUNK)]
            rvec = idx_r[pl.ds(c * CHUNK, CHUNK)]
            acc = jnp.zeros((16,), jnp.float32)
            for j in range(CHUNK):
                sh = lax.bitwise_and(hvec[j], 7)
                st = lax.bitwise_and(tvec[j], 7)
                sr = lax.bitwise_and(rvec[j], 7)
                hv = [h_buf[b, j, sh, pl.ds(16 * q, 16)] for q in range(4)]
                tv = [t_buf[b, j, st, pl.ds(16 * q, 16)] for q in range(4)]
                rv = [r_buf[b, j, sr, pl.ds(16 * q, 16)] for q in range(4)]

                def inv_norm(vs):
                    ssq = ((vs[0] * vs[0] + vs[1] * vs[1])
                           + (vs[2] * vs[2] + vs[3] * vs[3]))
                    return _rsqrt_nr(jnp.maximum(_lane_sum(ssq, perms), 1e-24))

                ih = inv_norm(hv)
                it = inv_norm(tv)
                ir = inv_norm(rv)

                s = jnp.zeros((16,), jnp.float32)
                for q in range(4):
                    s = s + jnp.abs(hv[q] * ih + rv[q] * ir - tv[q] * it)
                score = _lane_sum(s, perms)
                acc = jnp.where(iota16 == j, score, acc)
            out_v[pl.ds(c * CHUNK, 16)] = acc

        def pair_body(g, _):
            c0 = 2 * g
            cps_a = fire(c0, 0)
            cps_b = fire(c0 + 1, 1)
            for cp in cps_a:
                cp.wait()
            compute(c0, 0)
            for cp in cps_b:
                cp.wait()
            compute(c0 + 1, 1)
            return 0

        lax.fori_loop(0, PAIRS, pair_body, 0)

        pltpu.sync_copy(out_v, out_hbm.at[pl.ds(base, B_PER_W)])

    return k(batch_h, batch_t, batch_r, ent_emb, rel_emb)


def kernel(batch_h, batch_t, batch_r, ent_emb, rel_emb):
    return _transe_sc(batch_h, batch_t, batch_r, ent_emb, rel_emb)
